# baseline (speedup 1.0000x reference)
import jax
import jax.numpy as jnp
from jax import lax
from jax.experimental import pallas as pl
from jax.experimental.pallas import tpu as pltpu
from jax.experimental.pallas import tpu_sc as plsc
from jax.experimental.layout import Layout, with_layout_constraint

NUM_CORES = 2
NUM_SUBCORES = 16
NUM_WORKERS = NUM_CORES * NUM_SUBCORES
G = 128
NBUF = 4
EMB = 64
PADDED = 128
SUBPAD = 56


VBLK = 128
VOCAB = 1000000
NFULL = VOCAB // VBLK
VPAD = (NFULL + 1) * VBLK
L = 16


def _tr_body(wt_hbm, wtail_hbm, tab_hbm, sbufs, obufs, isems, osems):
    c = lax.axis_index("c")
    s = lax.axis_index("s")
    wid = s * NUM_CORES + c
    n = 244 + jnp.where(wid < 4, 1, 0)
    g0 = wid * 244 + jnp.minimum(wid, 4)

    @pl.when(wid == NUM_WORKERS - 1)
    def _():
        pltpu.sync_copy(wtail_hbm, obufs[0])
        pltpu.sync_copy(obufs[0], tab_hbm.at[pl.ds(NFULL * VBLK, VBLK)])

    def fire_stage(t, b):
        c0 = (g0 + t) * VBLK
        pltpu.async_copy(
            wt_hbm.at[:, pl.ds(c0, VBLK)],
            sbufs[b].at[:, pl.ds(0, VBLK)], isems[b],
        )

    def wait_stage(b):
        pltpu.make_async_copy(
            wt_hbm.at[:, pl.ds(0, VBLK)],
            sbufs[b].at[:, pl.ds(0, VBLK)], isems[b],
        ).wait()

    def fire_write(t, b):
        pltpu.async_copy(obufs[b], tab_hbm.at[pl.ds((g0 + t) * VBLK, VBLK)], osems[b])

    def wait_write(b):
        pltpu.make_async_copy(
            obufs[b], tab_hbm.at[pl.ds(0, VBLK)], osems[b]
        ).wait()

    iota = lax.iota(jnp.int32, L)
    dvecs = [jnp.arange(L, dtype=jnp.int32) + d0 for d0 in range(0, EMB, L)]

    def transpose_block(b):
        sbuf = sbufs[b]
        obuf = obufs[b]

        def row_pair(r, carry):
            rvecs = [jnp.full((L,), 2 * r + h, dtype=jnp.int32) for h in range(2)]
            vs = [
                plsc.load_gather(sbuf, [dvecs[k], rvecs[h]])
                for h in range(2)
                for k in range(EMB // L)
            ]
            i = 0
            for h in range(2):
                for k in range(EMB // L):
                    obuf[2 * r + h, pl.ds(k * L, L)] = vs[i]
                    i += 1
            return carry

        lax.fori_loop(0, VBLK // 2, row_pair, 0)

    fire_stage(0, 0)
    fire_stage(1, 1)
    for t in (0, 1):
        b = t % 2
        wait_stage(b)
        transpose_block(b)
        fire_write(t, b)
        fire_stage(t + 2, b)

    def step_pair(r, carry):
        for b in (0, 1):
            t = 2 + 2 * r + b
            wait_stage(b)
            wait_write(b)
            transpose_block(b)
            fire_write(t, b)

            @pl.when(t + 2 < n)
            def _():
                fire_stage(t + 2, b)
        return carry

    lax.fori_loop(0, 121, step_pair, 0)

    @pl.when(wid < 4)
    def _():
        wait_stage(0)
        wait_write(0)
        transpose_block(0)
        fire_write(244, 0)

    wait_write(0)
    wait_write(1)


def _transpose_pad(wt, wtail):
    mesh = plsc.VectorSubcoreMesh(core_axis_name="c", subcore_axis_name="s")
    return pl.kernel(
        _tr_body,
        out_type=jax.ShapeDtypeStruct((VPAD, PADDED), jnp.float32),
        mesh=mesh,
        scratch_types=[
            [pltpu.VMEM((EMB, VBLK + 1), jnp.float32) for _ in range(2)],
            [pltpu.VMEM((VBLK, PADDED), jnp.float32) for _ in range(2)],
            [pltpu.SemaphoreType.DMA for _ in range(2)],
            [pltpu.SemaphoreType.DMA for _ in range(2)],
        ],
        compiler_params=pltpu.CompilerParams(needs_layout_passes=False),
    )(wt, wtail)


def _body(idx_hbm, drow_hbm, table_hbm, out_hbm, idx_v, drow_v, rows, gsems, wsems):
    c = lax.axis_index("c")
    s = lax.axis_index("s")
    wid = s * NUM_CORES + c
    n_grp = idx_hbm.shape[0] // NUM_WORKERS
    base = wid * n_grp
    pltpu.sync_copy(idx_hbm.at[pl.ds(base, n_grp)], idx_v)
    pltpu.sync_copy(drow_hbm.at[pl.ds(base, n_grp)], drow_v)

    def fire_gather(j, b):
        pltpu.async_copy(table_hbm.at[idx_v.at[j]], rows[b], gsems[b])

    def fire_write(j, b):
        pltpu.async_copy(rows[b], out_hbm.at[drow_v.at[j]], wsems[b])

    def wait_g(b):
        pltpu.make_async_copy(table_hbm.at[pl.ds(0, G)], rows[b], gsems[b]).wait()

    def wait_w(b):
        pltpu.make_async_copy(rows[b], out_hbm.at[pl.ds(0, G)], wsems[b]).wait()

    for j in range(NBUF):
        fire_gather(j, j % NBUF)
        if j >= NBUF - 1:
            jj = j - (NBUF - 1)
            wait_g(jj % NBUF)
            fire_write(jj, jj % NBUF)

    def round_body(r, carry):
        j0 = NBUF + r * NBUF
        for b in range(NBUF):
            j = j0 + b
            wait_w(b)
            fire_gather(j, b)
            bb = (b + 1) % NBUF
            wait_g(bb)
            fire_write(j - (NBUF - 1), bb)
        return carry

    n_rounds = (n_grp - NBUF) // NBUF
    lax.fori_loop(0, n_rounds, round_body, 0)

    for j in range(n_grp, n_grp + NBUF - 1):
        jj = j - (NBUF - 1)
        wait_g(jj % NBUF)
        fire_write(jj, jj % NBUF)
    for b in range(NBUF):
        wait_w(b)


@jax.jit
def kernel(token_ids, weights):
    B, S = token_ids.shape
    n = B * S
    n_grp_total = n // G
    idx = token_ids.reshape(n_grp_total, G).astype(jnp.int32)
    e = jnp.arange(n, dtype=jnp.int32)
    drow = ((e // S) * SUBPAD + e % S).reshape(n_grp_total, G)
    wtail = jnp.pad(weights[NFULL * VBLK:], ((0, VBLK - (VOCAB - NFULL * VBLK)), (0, PADDED - EMB)))
    wpad = _transpose_pad(weights.T, wtail)
    mesh = plsc.VectorSubcoreMesh(core_axis_name="c", subcore_axis_name="s")
    n_grp = n_grp_total // NUM_WORKERS
    out = pl.kernel(
        _body,
        out_type=jax.ShapeDtypeStruct((B * SUBPAD, PADDED), jnp.float32),
        mesh=mesh,
        scratch_types=[
            pltpu.VMEM((n_grp, G), jnp.int32),
            pltpu.VMEM((n_grp, G), jnp.int32),
            [pltpu.VMEM((G, PADDED), jnp.float32) for _ in range(NBUF)],
            [pltpu.SemaphoreType.DMA for _ in range(NBUF)],
            [pltpu.SemaphoreType.DMA for _ in range(NBUF)],
        ],
    )(idx, drow, wpad)
    out3 = out.reshape(B, SUBPAD, PADDED)[:, :S, :EMB]
    return with_layout_constraint(out3, Layout(major_to_minor=(0, 1, 2)))


# speedup vs baseline: 1.5165x; 1.5165x over previous
import jax
import jax.numpy as jnp
from jax import lax
from jax.experimental import pallas as pl
from jax.experimental.pallas import tpu as pltpu
from jax.experimental.pallas import tpu_sc as plsc
from jax.experimental.layout import Layout, with_layout_constraint

NUM_CORES = 2
NUM_SUBCORES = 16
NUM_WORKERS = NUM_CORES * NUM_SUBCORES
G = 128
NBUF = 4
EMB = 64
PADDED = 128
SUBPAD = 56


def _body(idx_hbm, drow_hbm, table_hbm, out_hbm, idx_v, drow_v, rows, gsems, wsems):
    c = lax.axis_index("c")
    s = lax.axis_index("s")
    wid = s * NUM_CORES + c
    n_grp = idx_hbm.shape[0] // NUM_WORKERS
    base = wid * n_grp
    pltpu.sync_copy(idx_hbm.at[pl.ds(base, n_grp)], idx_v)
    pltpu.sync_copy(drow_hbm.at[pl.ds(base, n_grp)], drow_v)

    def fire_gather(j, b):
        pltpu.async_copy(table_hbm.at[idx_v.at[j]], rows[b], gsems[b])

    def fire_write(j, b):
        pltpu.async_copy(rows[b], out_hbm.at[drow_v.at[j]], wsems[b])

    def wait_g(b):
        pltpu.make_async_copy(table_hbm.at[pl.ds(0, G)], rows[b], gsems[b]).wait()

    def wait_w(b):
        pltpu.make_async_copy(rows[b], out_hbm.at[pl.ds(0, G)], wsems[b]).wait()

    for j in range(NBUF):
        fire_gather(j, j % NBUF)
        if j >= NBUF - 1:
            jj = j - (NBUF - 1)
            wait_g(jj % NBUF)
            fire_write(jj, jj % NBUF)

    def round_body(r, carry):
        j0 = NBUF + r * NBUF
        for b in range(NBUF):
            j = j0 + b
            wait_w(b)
            fire_gather(j, b)
            bb = (b + 1) % NBUF
            wait_g(bb)
            fire_write(j - (NBUF - 1), bb)
        return carry

    n_rounds = (n_grp - NBUF) // NBUF
    lax.fori_loop(0, n_rounds, round_body, 0)

    for j in range(n_grp, n_grp + NBUF - 1):
        jj = j - (NBUF - 1)
        wait_g(jj % NBUF)
        fire_write(jj, jj % NBUF)
    for b in range(NBUF):
        wait_w(b)


@jax.jit
def kernel(token_ids, weights):
    B, S = token_ids.shape
    n = B * S
    n_grp_total = n // G
    idx = token_ids.reshape(n_grp_total, G).astype(jnp.int32)
    e = jnp.arange(n, dtype=jnp.int32)
    drow = ((e // S) * SUBPAD + e % S).reshape(n_grp_total, G)
    wpad = jnp.pad(weights, ((0, 0), (0, PADDED - EMB)))
    mesh = plsc.VectorSubcoreMesh(core_axis_name="c", subcore_axis_name="s")
    n_grp = n_grp_total // NUM_WORKERS
    out = pl.kernel(
        _body,
        out_type=jax.ShapeDtypeStruct((B * SUBPAD, PADDED), jnp.float32),
        mesh=mesh,
        scratch_types=[
            pltpu.VMEM((n_grp, G), jnp.int32),
            pltpu.VMEM((n_grp, G), jnp.int32),
            [pltpu.VMEM((G, PADDED), jnp.float32) for _ in range(NBUF)],
            [pltpu.SemaphoreType.DMA for _ in range(NBUF)],
            [pltpu.SemaphoreType.DMA for _ in range(NBUF)],
        ],
    )(idx, drow, wpad)
    out3 = out.reshape(B, SUBPAD, PADDED)[:, :S, :EMB]
    return with_layout_constraint(out3, Layout(major_to_minor=(0, 1, 2)))
